# baseline (speedup 1.0000x reference)
import functools

import jax
import jax.numpy as jnp
from jax import lax
from jax.experimental import pallas as pl
from jax.experimental.pallas import tpu as pltpu
from jax.experimental.pallas import tpu_sc as plsc

B = 16384
V = 1000000
R = 64
RP = 128
ALPHA = 0.01

NC = 2
NS = 16
NW = NC * NS
BPW = B // NW
CHUNK = 128
NCHUNK = BPW // CHUNK


def _sc_gather_body(kr_hbm, emb0_hbm, emb1_hbm, e0_hbm, e1_hbm,
                    idx_v, rows0_v, rows1_v, sem0, sem1):
    wid = lax.axis_index("s") * NC + lax.axis_index("c")
    base = wid * BPW
    r0 = wid * NCHUNK
    pltpu.sync_copy(kr_hbm.at[pl.ds(r0, NCHUNK)], idx_v.at[pl.ds(0, NCHUNK)])
    pltpu.sync_copy(kr_hbm.at[pl.ds(B // CHUNK + r0, NCHUNK)],
                    idx_v.at[pl.ds(NCHUNK, NCHUNK)])
    copies = []
    for j in range(NCHUNK):
        copies.append(pltpu.async_copy(
            emb0_hbm.at[idx_v.at[j]],
            rows0_v.at[pl.ds(j * CHUNK, CHUNK)], sem0))
    for j in range(NCHUNK):
        copies.append(pltpu.async_copy(
            emb1_hbm.at[idx_v.at[NCHUNK + j]],
            rows1_v.at[pl.ds(j * CHUNK, CHUNK)], sem1))
    for c in copies:
        c.wait()
    pltpu.sync_copy(rows0_v, e0_hbm.at[pl.ds(base, BPW)])
    pltpu.sync_copy(rows1_v, e1_hbm.at[pl.ds(base, BPW)])


@functools.lru_cache(maxsize=None)
def _make_sc_gather():
    return pl.kernel(
        _sc_gather_body,
        out_type=(jax.ShapeDtypeStruct((B, R), jnp.float32),
                  jax.ShapeDtypeStruct((B, R), jnp.float32)),
        mesh=plsc.VectorSubcoreMesh(core_axis_name="c", subcore_axis_name="s",
                                    num_cores=NC, num_subcores=NS),
        scratch_types=[
            pltpu.VMEM((2 * NCHUNK, CHUNK), jnp.int32),
            pltpu.VMEM((BPW, R), jnp.float32),
            pltpu.VMEM((BPW, R), jnp.float32),
            pltpu.SemaphoreType.DMA,
            pltpu.SemaphoreType.DMA,
        ],
        compiler_params=pltpu.CompilerParams(use_tc_tiling_on_sc=False),
    )


def _dense_body(ll_ref, e0_ref, e1_ref, w0_ref, a0_ref, a1_ref, b1_ref,
                wmu_ref, bmu_ref, wlv_ref, blv_ref, mu_ref, lv_ref):
    e0 = e0_ref[...]
    e1 = e1_ref[...]
    h = (ll_ref[...] * w0_ref[...]
         + jnp.dot(e0, a0_ref[...], preferred_element_type=jnp.float32)
         + jnp.dot(e1, a1_ref[...], preferred_element_type=jnp.float32)
         + b1_ref[...])
    h = jnp.where(h >= 0, h, ALPHA * h)
    mu_ref[...] = (jnp.dot(h, wmu_ref[...], preferred_element_type=jnp.float32)
                   + bmu_ref[...] + e0 + e1)
    lv_ref[...] = (jnp.dot(h, wlv_ref[...], preferred_element_type=jnp.float32)
                   + blv_ref[...])


def _dense(ll, e0, e1, w0, a0, a1, b1, wmu, bmu, wlv, blv, blk=2048):
    grid = B // blk
    row_spec = lambda w: pl.BlockSpec((blk, w), lambda i: (i, 0))
    full = lambda s: pl.BlockSpec(s, lambda i: (0, 0))
    return pl.pallas_call(
        _dense_body,
        grid=(grid,),
        in_specs=[
            row_spec(1), row_spec(R), row_spec(R),
            full((1, RP)), full((R, RP)), full((R, RP)), full((1, RP)),
            full((RP, R)), full((1, R)), full((RP, R)), full((1, R)),
        ],
        out_specs=[row_spec(R), row_spec(R)],
        out_shape=[jax.ShapeDtypeStruct((B, R), jnp.float32),
                   jax.ShapeDtypeStruct((B, R), jnp.float32)],
    )(ll, e0, e1, w0, a0, a1, b1, wmu, bmu, wlv, blv)


def kernel(log_lib, K, emb0, emb1, W1, b1, Wmu, bmu, Wlv, blv):
    kr = K.reshape(2 * (B // CHUNK), CHUNK)
    e0, e1 = _make_sc_gather()(kr, emb0, emb1)
    w0 = W1[:, 0:1].T
    a0 = W1[:, 1:1 + R].T
    a1 = W1[:, 1 + R:1 + 2 * R].T
    mu, lv = _dense(log_lib.reshape(B, 1), e0, e1, w0, a0, a1,
                    b1.reshape(1, RP), Wmu.T, bmu.reshape(1, R),
                    Wlv.T, blv.reshape(1, R))
    return mu, lv
